# baseline (speedup 1.0000x reference)
import functools

import jax
import jax.numpy as jnp
from jax import lax
from jax.experimental import pallas as pl
from jax.experimental.pallas import tpu as pltpu
from jax.experimental.pallas import tpu_sc as plsc

N_DIM = 32
BATCH = 16384
NC = 2
NS = 16
NW = NC * NS
B_PER_W = BATCH // NW
IDX_CHUNK = 128
N_CHUNKS = B_PER_W // IDX_CHUNK
LANES = 16
N_GROUPS = B_PER_W // LANES


def _mf_body(u_hbm, i_hbm, ub_hbm, uv_hbm, ib_hbm, iv_hbm, gb_hbm, out_hbm,
             u_idx, i_idx, vu, vi, bu, bi, out_v, gv, sem):
    wid = lax.axis_index("s") * NC + lax.axis_index("c")
    base = wid * B_PER_W

    for c in range(N_CHUNKS):
        pltpu.sync_copy(u_hbm.at[pl.ds(base + c * IDX_CHUNK, IDX_CHUNK)],
                        u_idx.at[c])
        pltpu.sync_copy(i_hbm.at[pl.ds(base + c * IDX_CHUNK, IDX_CHUNK)],
                        i_idx.at[c])
    pltpu.sync_copy(gb_hbm, gv)
    gvec = gv[...]

    copies = []
    for c in range(N_CHUNKS):
        lo = c * IDX_CHUNK
        copies.append(pltpu.async_copy(
            uv_hbm.at[u_idx.at[c]], vu.at[pl.ds(lo, IDX_CHUNK)], sem))
        copies.append(pltpu.async_copy(
            iv_hbm.at[i_idx.at[c]], vi.at[pl.ds(lo, IDX_CHUNK)], sem))
        copies.append(pltpu.async_copy(
            ub_hbm.at[u_idx.at[c]], bu.at[pl.ds(lo, IDX_CHUNK)], sem))
        copies.append(pltpu.async_copy(
            ib_hbm.at[i_idx.at[c]], bi.at[pl.ds(lo, IDX_CHUNK)], sem))
    for cp in copies:
        cp.wait()

    lanes = lax.iota(jnp.int32, LANES)

    def group(gi, carry):
        row = gi * LANES
        acc = bu[pl.ds(row, LANES)] + bi[pl.ds(row, LANES)] + gvec
        ridx = row + lanes
        for d in range(N_DIM):
            didx = jnp.full((LANES,), d, jnp.int32)
            acc = acc + (plsc.load_gather(vu, [ridx, didx]) *
                         plsc.load_gather(vi, [ridx, didx]))
        out_v[pl.ds(row, LANES)] = acc
        return carry

    lax.fori_loop(0, N_GROUPS, group, 0)

    pltpu.sync_copy(out_v, out_hbm.at[pl.ds(base, B_PER_W)])


_mf = functools.partial(
    pl.kernel,
    mesh=plsc.VectorSubcoreMesh(core_axis_name="c", subcore_axis_name="s"),
    compiler_params=pltpu.CompilerParams(
        needs_layout_passes=False, use_tc_tiling_on_sc=False),
    out_type=jax.ShapeDtypeStruct((BATCH,), jnp.float32),
    scratch_types=[
        pltpu.VMEM((N_CHUNKS, IDX_CHUNK), jnp.int32),
        pltpu.VMEM((N_CHUNKS, IDX_CHUNK), jnp.int32),
        pltpu.VMEM((B_PER_W, N_DIM), jnp.float32),
        pltpu.VMEM((B_PER_W, N_DIM), jnp.float32),
        pltpu.VMEM((B_PER_W,), jnp.float32),
        pltpu.VMEM((B_PER_W,), jnp.float32),
        pltpu.VMEM((B_PER_W,), jnp.float32),
        pltpu.VMEM((LANES,), jnp.float32),
        pltpu.SemaphoreType.DMA,
    ],
)(_mf_body)


@jax.jit
def kernel(u, i, user_bias, user_vec, item_bias, item_vec, glob_bias):
    u = u.astype(jnp.int32)
    i = i.astype(jnp.int32)
    gb = jnp.broadcast_to(glob_bias.reshape(()), (LANES,))
    return _mf(u, i, user_bias, user_vec, item_bias, item_vec, gb)
